# Initial kernel scaffold; baseline (speedup 1.0000x reference)
import jax
import jax.numpy as jnp
from jax.experimental import pallas as pl


def kernel(input_sentence, lengths, word_embeddings):
    raise NotImplementedError("write your pallas kernel here")


# speedup vs baseline: 1.0939x; 1.0939x over previous
import jax
import jax.numpy as jnp
from jax import lax
from jax.experimental import pallas as pl
from jax.experimental.pallas import tpu as pltpu
from jax.experimental.pallas import tpu_sc as plsc

_PAD = 1
_NC = 2
_NS = 16
_NW = _NC * _NS
_LANES = 16
_NACC = 4


def _cbow_body(idx_hbm, len_hbm, emb_hbm, out_hbm,
               idx_v, acc_v, out_v, len_v, cnt_v, rlen_v, emb1_v, sems):
    L, B = idx_hbm.shape
    _, E = emb_hbm.shape
    bpw = B // _NW
    nj = bpw // _LANES
    ev = E // _LANES
    wid = lax.axis_index("s") * _NC + lax.axis_index("c")
    base = wid * bpw

    pltpu.sync_copy(len_hbm.at[pl.ds(base, bpw)], len_v)
    pltpu.sync_copy(emb_hbm.at[pl.ds(_PAD, 1)], emb1_v)
    pltpu.sync_copy(idx_hbm.at[:, pl.ds(base, bpw)], idx_v)

    for a in range(_NACC):
        pltpu.make_async_copy(
            emb_hbm.at[idx_v.at[a]], acc_v.at[a], sems.at[a]).start()

    def step(g, carry):
        for a in range(_NACC):
            l = g * _NACC + a
            pltpu.make_async_copy(
                emb_hbm.at[idx_v.at[l - _NACC]], acc_v.at[a],
                sems.at[a]).wait()
            pltpu.make_async_copy(
                emb_hbm.at[idx_v.at[l]], acc_v.at[a],
                sems.at[a]).start(add=True)
        return carry
    lax.fori_loop(1, L // _NACC, step, 0)

    def cbody(l, carry):
        return tuple(
            c + (idx_v[l, pl.ds(j * _LANES, _LANES)] == _PAD)
            .astype(jnp.int32)
            for j, c in enumerate(carry))
    cnt = lax.fori_loop(
        0, L, cbody,
        tuple(jnp.zeros((_LANES,), jnp.int32) for _ in range(nj)))
    for j in range(nj):
        ds = pl.ds(j * _LANES, _LANES)
        cnt_v[ds] = cnt[j].astype(jnp.float32)
        rlen_v[ds] = 1.0 / len_v[ds].astype(jnp.float32)

    for a in range(_NACC):
        l = L - _NACC + a
        pltpu.make_async_copy(
            emb_hbm.at[idx_v.at[l]], acc_v.at[l % _NACC],
            sems.at[l % _NACC]).wait()

    def obody(b, carry):
        bb = jnp.full((_LANES,), b, jnp.int32)
        c = plsc.load_gather(cnt_v, [bb])
        r = plsc.load_gather(rlen_v, [bb])
        for e in range(ev):
            ds = pl.ds(e * _LANES, _LANES)
            tot = ((acc_v[0, b, ds] + acc_v[1, b, ds])
                   + (acc_v[2, b, ds] + acc_v[3, b, ds]))
            out_v[b, ds] = (tot - c * emb1_v[0, ds]) * r
        return carry
    lax.fori_loop(0, bpw, obody, 0)

    pltpu.sync_copy(out_v, out_hbm.at[pl.ds(base, bpw)])


def kernel(input_sentence, lengths, word_embeddings):
    B, L = input_sentence.shape
    _, E = word_embeddings.shape
    bpw = B // _NW
    idx_t = input_sentence.astype(jnp.int32).T
    f = pl.kernel(
        _cbow_body,
        out_type=jax.ShapeDtypeStruct((B, E), jnp.float32),
        mesh=plsc.VectorSubcoreMesh(
            core_axis_name="c", subcore_axis_name="s",
            num_cores=_NC, num_subcores=_NS),
        scratch_types=[
            pltpu.VMEM((L, bpw), jnp.int32),
            pltpu.VMEM((_NACC, bpw, E), jnp.float32),
            pltpu.VMEM((bpw, E), jnp.float32),
            pltpu.VMEM((bpw,), jnp.int32),
            pltpu.VMEM((bpw,), jnp.float32),
            pltpu.VMEM((bpw,), jnp.float32),
            pltpu.VMEM((1, E), jnp.float32),
            pltpu.SemaphoreType.DMA((_NACC,)),
        ],
        compiler_params=pltpu.CompilerParams(
            use_tc_tiling_on_sc=False, needs_layout_passes=False),
    )
    return f(idx_t, lengths.astype(jnp.int32), word_embeddings)
